# Initial kernel scaffold; baseline (speedup 1.0000x reference)
import jax
import jax.numpy as jnp
from jax.experimental import pallas as pl


def kernel(x, edge_index, W, Wb, a_w, a_b):
    raise NotImplementedError("write your pallas kernel here")


# speedup vs baseline: 51.7777x; 51.7777x over previous
import functools

import jax
import jax.numpy as jnp
from jax import lax
from jax.experimental import pallas as pl
from jax.experimental.pallas import tpu as pltpu
from jax.experimental.pallas import tpu_sc as plsc

ALPHA = 0.2


def kernel(x, edge_index, W, Wb, a_w, a_b):
    N, D = x.shape
    H, _, DH = W.shape
    E = edge_index.shape[1]
    F = H * DH
    ROW = F + 16

    Wcat = jnp.transpose(W, (1, 0, 2)).reshape(D, F)
    bcat = Wb.reshape(1, F).astype(jnp.float32)
    eyeH = jnp.eye(H, dtype=jnp.float32)
    A_src = (a_w[:, :DH][:, :, None] * eyeH[:, None, :]).reshape(F, H)
    A_dst = (a_w[:, DH:][:, :, None] * eyeH[:, None, :]).reshape(F, H)
    Acat = jnp.concatenate([A_src, A_dst], axis=1)
    ab8 = jnp.concatenate([jnp.zeros((H,), jnp.float32), a_b]).reshape(1, 2 * H)

    def proj_body(x_ref, wc_ref, bc_ref, ac_ref, ab_ref, wh_ref, sp_ref):
        xv = x_ref[...]
        wh = lax.dot(xv, wc_ref[...], precision=lax.Precision.HIGHEST,
                     preferred_element_type=jnp.float32) + bc_ref[...]
        wh_ref[...] = wh
        sp_ref[...] = lax.dot(wh, ac_ref[...], precision=lax.Precision.HIGHEST,
                              preferred_element_type=jnp.float32) + ab_ref[...]

    whflat, spair = pl.pallas_call(
        proj_body,
        out_shape=(jax.ShapeDtypeStruct((N, F), jnp.float32),
                   jax.ShapeDtypeStruct((N, 2 * H), jnp.float32)),
    )(x, Wcat, bcat, Acat, ab8)

    NC, NS = 2, 16
    NW = NC * NS
    e_per_w = E // NW
    CH = 80
    n_chunks = e_per_w // CH
    rows_per_tile = N // NS
    nfull = rows_per_tile // CH
    rem = rows_per_tile - nfull * CH

    src = edge_index[0]
    dst = edge_index[1]

    mesh = plsc.VectorSubcoreMesh(core_axis_name="c", subcore_axis_name="s")

    @functools.partial(
        pl.kernel, mesh=mesh,
        compiler_params=pltpu.CompilerParams(use_tc_tiling_on_sc=False,
                                             needs_layout_passes=False),
        out_type=jax.ShapeDtypeStruct((NC, N, ROW), jnp.float32),
        scratch_types=[
            pltpu.VMEM((CH, F), jnp.float32),
            pltpu.VMEM((CH, ROW), jnp.float32),
            pltpu.VMEM((CH,), jnp.int32),
            pltpu.VMEM((CH,), jnp.int32),
            pltpu.VMEM((CH, 2 * H), jnp.float32),
            pltpu.VMEM((CH, 2 * H), jnp.float32),
            pltpu.VMEM_SHARED((N, 2 * H), jnp.float32),
            pltpu.VMEM_SHARED((N, ROW), jnp.float32),
            pltpu.SemaphoreType.DMA,
            pltpu.SemaphoreType.DMA,
            pltpu.SemaphoreType.DMA,
        ],
    )
    def sc_pass(src_hbm, dst_hbm, spair_hbm, wh_hbm, out_hbm,
                rows_v, staging, srcv, dstv, ssrc8, sdst8, spair_sh, acc,
                sem, sem2, sem3):
        cid = lax.axis_index("c")
        sid = lax.axis_index("s")
        wid = cid * NS + sid

        @pl.when(sid == 0)
        def _():
            pltpu.sync_copy(spair_hbm, spair_sh)

        zero16 = jnp.zeros((16,), jnp.float32)

        @pl.loop(0, CH)
        def _(r):
            for c in range(ROW // 16):
                staging[r, pl.ds(c * 16, 16)] = zero16

        r0 = sid * rows_per_tile

        @pl.loop(0, nfull)
        def _(i):
            pltpu.sync_copy(staging, acc.at[pl.ds(r0 + i * CH, CH)])

        if rem:
            pltpu.sync_copy(staging.at[pl.ds(0, rem)],
                            acc.at[pl.ds(r0 + nfull * CH, rem)])
        plsc.subcore_barrier()

        base = wid * e_per_w

        @pl.loop(0, n_chunks)
        def _(j):
            eb = base + j * CH
            pltpu.sync_copy(src_hbm.at[pl.ds(eb, CH)], srcv)
            pltpu.sync_copy(dst_hbm.at[pl.ds(eb, CH)], dstv)
            gdesc = pltpu.async_copy(wh_hbm.at[srcv], rows_v, sem)
            s1 = pltpu.async_copy(spair_sh.at[srcv], ssrc8, sem2)
            s2 = pltpu.async_copy(spair_sh.at[dstv], sdst8, sem3)
            s1.wait()
            s2.wait()

            @pl.loop(0, CH // 16)
            def _(g):
                iota = lax.iota(jnp.int32, 16)
                e_vec = g * 16 + iota
                for h in range(H):
                    a = plsc.load_gather(ssrc8, [e_vec, jnp.full((16,), h, jnp.int32)])
                    b = plsc.load_gather(sdst8, [e_vec, jnp.full((16,), H + h, jnp.int32)])
                    z = a + b
                    ex = jnp.exp(jnp.maximum(z, z * ALPHA))
                    plsc.store_scatter(staging,
                                       [e_vec, jnp.full((16,), F + h, jnp.int32)],
                                       ex)

            gdesc.wait()

            @pl.loop(0, CH)
            def _(e):
                exvec = staging[e, pl.ds(F, 16)]
                ss = tuple(exvec[h] for h in range(H))
                for c in range(F // 16):
                    staging[e, pl.ds(c * 16, 16)] = (
                        rows_v[e, pl.ds(c * 16, 16)] * ss[c // 2])

            pltpu.sync_copy(staging, acc.at[dstv], add=True)

        plsc.subcore_barrier()

        @pl.loop(0, nfull)
        def _(i):
            pltpu.sync_copy(acc.at[pl.ds(r0 + i * CH, CH)],
                            out_hbm.at[cid, pl.ds(r0 + i * CH, CH)])

        if rem:
            pltpu.sync_copy(acc.at[pl.ds(r0 + nfull * CH, rem)],
                            out_hbm.at[cid, pl.ds(r0 + nfull * CH, rem)])

    acc2 = sc_pass(src, dst, spair, whflat)

    def comb_body(a_ref, o_ref):
        s = a_ref[0] + a_ref[1]
        den = s[:, F:F + H]
        r = 1.0 / (den + 1e-16)
        parts = [s[:, h * DH:(h + 1) * DH] * r[:, h:h + 1] for h in range(H)]
        o_ref[...] = jnp.concatenate(parts, axis=1)

    hp = pl.pallas_call(
        comb_body,
        out_shape=jax.ShapeDtypeStruct((N, F), jnp.float32),
    )(acc2)
    return hp


# speedup vs baseline: 96.5264x; 1.8642x over previous
import functools

import jax
import jax.numpy as jnp
from jax import lax
from jax.experimental import pallas as pl
from jax.experimental.pallas import tpu as pltpu
from jax.experimental.pallas import tpu_sc as plsc

ALPHA = 0.2


def kernel(x, edge_index, W, Wb, a_w, a_b):
    N, D = x.shape
    H, _, DH = W.shape
    E = edge_index.shape[1]
    F = H * DH
    SW = 16

    Wcat = jnp.transpose(W, (1, 0, 2)).reshape(D, F)
    bcat = Wb.reshape(1, F).astype(jnp.float32)
    eyeH = jnp.eye(H, dtype=jnp.float32)
    A_src = (a_w[:, :DH][:, :, None] * eyeH[:, None, :]).reshape(F, H)
    A_dst = (a_w[:, DH:][:, :, None] * eyeH[:, None, :]).reshape(F, H)
    Acat = jnp.concatenate(
        [A_src, A_dst, jnp.zeros((F, SW - 2 * H), jnp.float32)], axis=1)
    ab16 = jnp.concatenate(
        [jnp.zeros((H,), jnp.float32), a_b,
         jnp.zeros((SW - 2 * H,), jnp.float32)]).reshape(1, SW)

    def proj_body(x_ref, wc_ref, bc_ref, ac_ref, ab_ref, wh_ref, sp_ref):
        xv = x_ref[...]
        wh = lax.dot(xv, wc_ref[...], precision=lax.Precision.HIGHEST,
                     preferred_element_type=jnp.float32) + bc_ref[...]
        wh_ref[...] = wh
        sp_ref[...] = lax.dot(wh, ac_ref[...], precision=lax.Precision.HIGHEST,
                              preferred_element_type=jnp.float32) + ab_ref[...]

    whflat, s16 = pl.pallas_call(
        proj_body,
        out_shape=(jax.ShapeDtypeStruct((N, F), jnp.float32),
                   jax.ShapeDtypeStruct((N, SW), jnp.float32)),
    )(x, Wcat, bcat, Acat, ab16)

    NC, NS = 2, 16
    NW = NC * NS
    e_per_w = E // NW
    CH = 80
    n_chunks = e_per_w // CH
    rows_per_tile = N // NS
    nfull = rows_per_tile // CH
    rem = rows_per_tile - nfull * CH

    src = edge_index[0]
    dst = edge_index[1]

    mesh = plsc.VectorSubcoreMesh(core_axis_name="c", subcore_axis_name="s")

    @functools.partial(
        pl.kernel, mesh=mesh,
        compiler_params=pltpu.CompilerParams(use_tc_tiling_on_sc=False,
                                             needs_layout_passes=False),
        out_type=(jax.ShapeDtypeStruct((NC, N, F), jnp.float32),
                  jax.ShapeDtypeStruct((NC, N, SW), jnp.float32)),
        scratch_types=[
            pltpu.VMEM((2, CH, F), jnp.float32),
            pltpu.VMEM((2, CH, SW), jnp.float32),
            pltpu.VMEM((2, CH, SW), jnp.float32),
            pltpu.VMEM((2, CH, SW), jnp.float32),
            pltpu.VMEM((2, CH), jnp.int32),
            pltpu.VMEM((2, CH), jnp.int32),
            pltpu.VMEM_SHARED((N, F), jnp.float32),
            pltpu.VMEM_SHARED((N, SW), jnp.float32),
            pltpu.SemaphoreType.DMA((2,)),
            pltpu.SemaphoreType.DMA((2,)),
            pltpu.SemaphoreType.DMA((2,)),
            pltpu.SemaphoreType.DMA((2,)),
            pltpu.SemaphoreType.DMA((2,)),
        ],
    )
    def sc_pass(src_hbm, dst_hbm, s16_hbm, wh_hbm, oacc_hbm, oden_hbm,
                rows_v, exb, ssrc16, sdst16, srcv, dstv, acc, accden,
                gsem_r, gsem_s, gsem_d, ssem_r, ssem_d):
        cid = lax.axis_index("c")
        sid = lax.axis_index("s")
        wid = cid * NS + sid
        base = wid * e_per_w
        zero16 = jnp.zeros((16,), jnp.float32)

        @pl.loop(0, CH)
        def _(r):
            for c in range(F // 16):
                rows_v[0, r, pl.ds(c * 16, 16)] = zero16
            exb[0, r, :] = zero16
            exb[1, r, :] = zero16

        r0 = sid * rows_per_tile

        @pl.loop(0, nfull)
        def _(i):
            pltpu.sync_copy(rows_v.at[0], acc.at[pl.ds(r0 + i * CH, CH)])
            pltpu.sync_copy(exb.at[0], accden.at[pl.ds(r0 + i * CH, CH)])

        if rem:
            pltpu.sync_copy(rows_v.at[0, pl.ds(0, rem)],
                            acc.at[pl.ds(r0 + nfull * CH, rem)])
            pltpu.sync_copy(exb.at[0, pl.ds(0, rem)],
                            accden.at[pl.ds(r0 + nfull * CH, rem)])
        plsc.subcore_barrier()

        def load_idx(b, j):
            eb = base + j * CH
            pltpu.sync_copy(src_hbm.at[pl.ds(eb, CH)], srcv.at[b])
            pltpu.sync_copy(dst_hbm.at[pl.ds(eb, CH)], dstv.at[b])

        def issue_gathers(b):
            pltpu.async_copy(wh_hbm.at[srcv.at[b]], rows_v.at[b], gsem_r.at[b])
            pltpu.async_copy(s16_hbm.at[srcv.at[b]], ssrc16.at[b], gsem_s.at[b])
            pltpu.async_copy(s16_hbm.at[dstv.at[b]], sdst16.at[b], gsem_d.at[b])

        def wait_gathers(b):
            pltpu.make_async_copy(wh_hbm.at[srcv.at[b]], rows_v.at[b],
                                  gsem_r.at[b]).wait()
            pltpu.make_async_copy(s16_hbm.at[srcv.at[b]], ssrc16.at[b],
                                  gsem_s.at[b]).wait()
            pltpu.make_async_copy(s16_hbm.at[dstv.at[b]], sdst16.at[b],
                                  gsem_d.at[b]).wait()

        def wait_scatters(b):
            pltpu.make_async_copy(rows_v.at[b], acc.at[dstv.at[b]],
                                  ssem_r.at[b]).wait()
            pltpu.make_async_copy(exb.at[b], accden.at[dstv.at[b]],
                                  ssem_d.at[b]).wait()

        def compute_and_scatter(b):
            @pl.loop(0, CH // 16)
            def _(g):
                iota = lax.iota(jnp.int32, 16)
                e_vec = g * 16 + iota
                for h in range(H):
                    a = plsc.load_gather(ssrc16.at[b],
                                         [e_vec, jnp.full((16,), h, jnp.int32)])
                    bb = plsc.load_gather(sdst16.at[b],
                                          [e_vec, jnp.full((16,), H + h, jnp.int32)])
                    z = a + bb
                    ex = jnp.exp(jnp.maximum(z, z * ALPHA))
                    plsc.store_scatter(exb.at[b],
                                       [e_vec, jnp.full((16,), h, jnp.int32)], ex)

            @pl.loop(0, CH)
            def _(e):
                exvec = exb[b, e, :]
                ss = tuple(exvec[h] for h in range(H))
                for c in range(F // 16):
                    rows_v[b, e, pl.ds(c * 16, 16)] = (
                        rows_v[b, e, pl.ds(c * 16, 16)] * ss[c // 2])

            pltpu.async_copy(rows_v.at[b], acc.at[dstv.at[b]], ssem_r.at[b],
                             add=True)
            pltpu.async_copy(exb.at[b], accden.at[dstv.at[b]], ssem_d.at[b],
                             add=True)

        load_idx(0, 0)
        issue_gathers(0)

        @pl.loop(0, n_chunks // 2)
        def _(jj):
            i0 = 2 * jj
            @pl.when(jj > 0)
            def _():
                wait_scatters(1)
            load_idx(1, i0 + 1)
            issue_gathers(1)
            wait_gathers(0)
            compute_and_scatter(0)
            @pl.when(i0 + 2 < n_chunks)
            def _():
                wait_scatters(0)
                load_idx(0, i0 + 2)
                issue_gathers(0)
            wait_gathers(1)
            compute_and_scatter(1)

        if n_chunks % 2:
            wait_gathers(0)
            compute_and_scatter(0)
            wait_scatters(0)
            wait_scatters(1)
        else:
            wait_scatters(0)
            wait_scatters(1)

        plsc.subcore_barrier()

        @pl.loop(0, nfull)
        def _(i):
            pltpu.sync_copy(acc.at[pl.ds(r0 + i * CH, CH)],
                            oacc_hbm.at[cid, pl.ds(r0 + i * CH, CH)])
            pltpu.sync_copy(accden.at[pl.ds(r0 + i * CH, CH)],
                            oden_hbm.at[cid, pl.ds(r0 + i * CH, CH)])

        if rem:
            pltpu.sync_copy(acc.at[pl.ds(r0 + nfull * CH, rem)],
                            oacc_hbm.at[cid, pl.ds(r0 + nfull * CH, rem)])
            pltpu.sync_copy(accden.at[pl.ds(r0 + nfull * CH, rem)],
                            oden_hbm.at[cid, pl.ds(r0 + nfull * CH, rem)])

    acc2, den2 = sc_pass(src, dst, s16, whflat)

    def comb_body(a_ref, d_ref, o_ref):
        s = a_ref[0] + a_ref[1]
        den = d_ref[0] + d_ref[1]
        r = 1.0 / (den[:, :H] + 1e-16)
        parts = [s[:, h * DH:(h + 1) * DH] * r[:, h:h + 1] for h in range(H)]
        o_ref[...] = jnp.concatenate(parts, axis=1)

    hp = pl.pallas_call(
        comb_body,
        out_shape=jax.ShapeDtypeStruct((N, F), jnp.float32),
    )(acc2, den2)
    return hp


# speedup vs baseline: 141.9407x; 1.4705x over previous
import functools

import jax
import jax.numpy as jnp
from jax import lax
from jax.experimental import pallas as pl
from jax.experimental.pallas import tpu as pltpu
from jax.experimental.pallas import tpu_sc as plsc

ALPHA = 0.2


def kernel(x, edge_index, W, Wb, a_w, a_b):
    N, D = x.shape
    H, _, DH = W.shape
    E = edge_index.shape[1]
    F = H * DH
    SW = 16

    Wcat = jnp.transpose(W, (1, 0, 2)).reshape(D, F)
    bcat = Wb.reshape(1, F).astype(jnp.float32)
    eyeH = jnp.eye(H, dtype=jnp.float32)
    A_src = (a_w[:, :DH][:, :, None] * eyeH[:, None, :]).reshape(F, H)
    A_dst = (a_w[:, DH:][:, :, None] * eyeH[:, None, :]).reshape(F, H)
    Acat = jnp.concatenate(
        [A_src, A_dst, jnp.zeros((F, SW - 2 * H), jnp.float32)], axis=1)
    ab16 = jnp.concatenate(
        [jnp.zeros((H,), jnp.float32), a_b,
         jnp.zeros((SW - 2 * H,), jnp.float32)]).reshape(1, SW)

    def proj_body(x_ref, wc_ref, bc_ref, ac_ref, ab_ref, wh_ref, sp_ref):
        xv = x_ref[...]
        wh = lax.dot(xv, wc_ref[...], precision=lax.Precision.HIGHEST,
                     preferred_element_type=jnp.float32) + bc_ref[...]
        wh_ref[...] = wh
        sp_ref[...] = lax.dot(wh, ac_ref[...], precision=lax.Precision.HIGHEST,
                              preferred_element_type=jnp.float32) + ab_ref[...]

    whflat, s16 = pl.pallas_call(
        proj_body,
        out_shape=(jax.ShapeDtypeStruct((N, F), jnp.float32),
                   jax.ShapeDtypeStruct((N, SW), jnp.float32)),
    )(x, Wcat, bcat, Acat, ab16)

    NC, NS = 2, 16
    NW = NC * NS
    e_per_w = E // NW
    CH = 80
    n_chunks = e_per_w // CH
    rows_per_tile = N // NS
    nfull = rows_per_tile // CH
    rem = rows_per_tile - nfull * CH

    src = edge_index[0]
    dst = edge_index[1]

    mesh = plsc.VectorSubcoreMesh(core_axis_name="c", subcore_axis_name="s")

    @functools.partial(
        pl.kernel, mesh=mesh,
        compiler_params=pltpu.CompilerParams(use_tc_tiling_on_sc=False,
                                             needs_layout_passes=False),
        out_type=(jax.ShapeDtypeStruct((NC, N, F), jnp.float32),
                  jax.ShapeDtypeStruct((NC, N, SW), jnp.float32)),
        scratch_types=[
            pltpu.VMEM((2, CH, F), jnp.float32),
            pltpu.VMEM((2, CH, SW), jnp.float32),
            pltpu.VMEM((2, CH, SW), jnp.float32),
            pltpu.VMEM((2, CH, SW), jnp.float32),
            pltpu.VMEM((2, CH), jnp.int32),
            pltpu.VMEM((2, CH), jnp.int32),
            pltpu.VMEM((2, CH), jnp.int32),
            pltpu.VMEM_SHARED((N, F), jnp.float32),
            pltpu.VMEM_SHARED((N, SW), jnp.float32),
            pltpu.SemaphoreType.DMA((2,)),
            pltpu.SemaphoreType.DMA((2,)),
            pltpu.SemaphoreType.DMA((2,)),
            pltpu.SemaphoreType.DMA((2,)),
            pltpu.SemaphoreType.DMA((2,)),
            pltpu.SemaphoreType.DMA((2,)),
            pltpu.SemaphoreType.DMA((2,)),
        ],
    )
    def sc_pass(src_hbm, dst_hbm, s16_hbm, wh_hbm, oacc_hbm, oden_hbm,
                rows_v, exb, ssrc16, sdst16, srcv, dstv, dstv_scat, acc, accden,
                gsem_r, gsem_s, gsem_d, ssem_r, ssem_d, isem_s, isem_d):
        cid = lax.axis_index("c")
        sid = lax.axis_index("s")
        wid = cid * NS + sid
        base = wid * e_per_w
        zero16 = jnp.zeros((16,), jnp.float32)

        @pl.loop(0, CH)
        def _(r):
            for c in range(F // 16):
                rows_v[0, r, pl.ds(c * 16, 16)] = zero16
            exb[0, r, :] = zero16
            exb[1, r, :] = zero16

        r0 = sid * rows_per_tile

        @pl.loop(0, nfull)
        def _(i):
            pltpu.sync_copy(rows_v.at[0], acc.at[pl.ds(r0 + i * CH, CH)])
            pltpu.sync_copy(exb.at[0], accden.at[pl.ds(r0 + i * CH, CH)])

        if rem:
            pltpu.sync_copy(rows_v.at[0, pl.ds(0, rem)],
                            acc.at[pl.ds(r0 + nfull * CH, rem)])
            pltpu.sync_copy(exb.at[0, pl.ds(0, rem)],
                            accden.at[pl.ds(r0 + nfull * CH, rem)])
        plsc.subcore_barrier()

        def issue_idx(b, j):
            eb = base + j * CH
            pltpu.async_copy(src_hbm.at[pl.ds(eb, CH)], srcv.at[b], isem_s.at[b])
            pltpu.async_copy(dst_hbm.at[pl.ds(eb, CH)], dstv.at[b], isem_d.at[b])

        def wait_idx(b):
            pltpu.make_async_copy(src_hbm.at[pl.ds(0, CH)], srcv.at[b],
                                  isem_s.at[b]).wait()
            pltpu.make_async_copy(dst_hbm.at[pl.ds(0, CH)], dstv.at[b],
                                  isem_d.at[b]).wait()

        def issue_gathers(b):
            pltpu.async_copy(wh_hbm.at[srcv.at[b]], rows_v.at[b], gsem_r.at[b])
            pltpu.async_copy(s16_hbm.at[srcv.at[b]], ssrc16.at[b], gsem_s.at[b])
            pltpu.async_copy(s16_hbm.at[dstv.at[b]], sdst16.at[b], gsem_d.at[b])

        def wait_gathers(b):
            pltpu.make_async_copy(wh_hbm.at[srcv.at[b]], rows_v.at[b],
                                  gsem_r.at[b]).wait()
            pltpu.make_async_copy(s16_hbm.at[srcv.at[b]], ssrc16.at[b],
                                  gsem_s.at[b]).wait()
            pltpu.make_async_copy(s16_hbm.at[dstv.at[b]], sdst16.at[b],
                                  gsem_d.at[b]).wait()

        def wait_scatters(b):
            pltpu.make_async_copy(rows_v.at[b], acc.at[dstv_scat.at[b]],
                                  ssem_r.at[b]).wait()
            pltpu.make_async_copy(exb.at[b], accden.at[dstv_scat.at[b]],
                                  ssem_d.at[b]).wait()

        def copy_dstv(b):
            @pl.loop(0, CH // 16)
            def _(g):
                dstv_scat[b, pl.ds(g * 16, 16)] = dstv[b, pl.ds(g * 16, 16)]

        def compute_and_scatter(b):
            @pl.loop(0, CH // 16)
            def _(g):
                iota = lax.iota(jnp.int32, 16)
                e_vec = g * 16 + iota
                av = [plsc.load_gather(ssrc16.at[b],
                                       [e_vec, jnp.full((16,), h, jnp.int32)])
                      for h in range(H)]
                bv = [plsc.load_gather(sdst16.at[b],
                                       [e_vec, jnp.full((16,), H + h, jnp.int32)])
                      for h in range(H)]
                zv = [a + bb for a, bb in zip(av, bv)]
                ev = [jnp.exp(jnp.maximum(z, z * ALPHA)) for z in zv]
                for h in range(H):
                    plsc.store_scatter(exb.at[b],
                                       [e_vec, jnp.full((16,), h, jnp.int32)],
                                       ev[h])

            @pl.loop(0, CH)
            def _(e):
                exvec = exb[b, e, :]
                ss = tuple(exvec[h] for h in range(H))
                for c in range(F // 16):
                    rows_v[b, e, pl.ds(c * 16, 16)] = (
                        rows_v[b, e, pl.ds(c * 16, 16)] * ss[c // 2])

            pltpu.async_copy(rows_v.at[b], acc.at[dstv_scat.at[b]],
                             ssem_r.at[b], add=True)
            pltpu.async_copy(exb.at[b], accden.at[dstv_scat.at[b]],
                             ssem_d.at[b], add=True)

        pltpu.async_copy(src_hbm.at[pl.ds(base, CH)], srcv.at[0], isem_s.at[0])
        pltpu.async_copy(dst_hbm.at[pl.ds(base, CH)], dstv.at[0], isem_d.at[0])
        issue_idx(1, 1)
        wait_idx(0)
        issue_gathers(0)

        @pl.loop(0, n_chunks // 2)
        def _(jj):
            i0 = 2 * jj
            wait_idx(1)

            @pl.when(jj > 0)
            def _():
                wait_scatters(1)
            issue_gathers(1)
            wait_gathers(0)
            copy_dstv(0)
            issue_idx(0, i0 + 2)
            compute_and_scatter(0)
            wait_idx(0)
            wait_scatters(0)
            issue_gathers(0)
            wait_gathers(1)
            copy_dstv(1)

            @pl.when(i0 + 3 < n_chunks)
            def _():
                issue_idx(1, i0 + 3)
            compute_and_scatter(1)

        if n_chunks % 2:
            wait_scatters(1)
            wait_gathers(0)
            copy_dstv(0)
            compute_and_scatter(0)
            wait_scatters(0)
        else:
            wait_scatters(0)
            wait_scatters(1)

        plsc.subcore_barrier()

        @pl.loop(0, nfull)
        def _(i):
            pltpu.sync_copy(acc.at[pl.ds(r0 + i * CH, CH)],
                            oacc_hbm.at[cid, pl.ds(r0 + i * CH, CH)])
            pltpu.sync_copy(accden.at[pl.ds(r0 + i * CH, CH)],
                            oden_hbm.at[cid, pl.ds(r0 + i * CH, CH)])

        if rem:
            pltpu.sync_copy(acc.at[pl.ds(r0 + nfull * CH, rem)],
                            oacc_hbm.at[cid, pl.ds(r0 + nfull * CH, rem)])
            pltpu.sync_copy(accden.at[pl.ds(r0 + nfull * CH, rem)],
                            oden_hbm.at[cid, pl.ds(r0 + nfull * CH, rem)])

    acc2, den2 = sc_pass(src, dst, s16, whflat)

    def comb_body(a_ref, d_ref, o_ref):
        s = a_ref[0] + a_ref[1]
        den = d_ref[0] + d_ref[1]
        r = 1.0 / (den[:, :H] + 1e-16)
        parts = [s[:, h * DH:(h + 1) * DH] * r[:, h:h + 1] for h in range(H)]
        o_ref[...] = jnp.concatenate(parts, axis=1)

    hp = pl.pallas_call(
        comb_body,
        out_shape=jax.ShapeDtypeStruct((N, F), jnp.float32),
    )(acc2, den2)
    return hp
